# baseline (speedup 1.0000x reference)
import functools

import jax
import jax.numpy as jnp
from jax import lax
from jax.experimental import pallas as pl
from jax.experimental.pallas import tpu as pltpu
from jax.experimental.pallas import tpu_sc as plsc

N = 10000
D = 128
NPAD = 10240
E = 320000
M = 2 * E
NC = 2
NS = 16
NW = NC * NS
K = 120
T = 56
CHUNKS = 3 * T
MSG_PER_W = K * CHUNKS
M_PAD = MSG_PER_W * NW
RPT = NPAD // NS

BR = 2048
GRID = NPAD // BR

_P = jax.lax.Precision.HIGHEST


def _mm_x_body(x_ref, w_ref, b_ref, out_ref):
    out_ref[...] = lax.dot_general(x_ref[...], w_ref[...],
                                   (((1,), (1,)), ((), ())),
                                   precision=_P) + b_ref[...]


def _mm_h_body(o_ref, p_ref, w_ref, b_ref, out_ref):
    h = jnp.maximum(o_ref[...] + p_ref[0] + p_ref[1], 0.0)
    out_ref[...] = lax.dot_general(h, w_ref[...], (((1,), (1,)), ((), ())),
                                   precision=_P) + b_ref[...]


def _fin_body(o_ref, p_ref, out_ref):
    out_ref[...] = o_ref[...] + p_ref[0] + p_ref[1]


_row_spec = pl.BlockSpec((BR, D), lambda i: (i, 0))
_pair_spec = pl.BlockSpec((2, BR, D), lambda i: (0, i, 0))
_w_spec = pl.BlockSpec((D, D), lambda i: (0, 0))
_b_spec = pl.BlockSpec((1, D), lambda i: (0, 0))
_out1 = jax.ShapeDtypeStruct((NPAD, D), jnp.float32)

_mm_x = pl.pallas_call(
    _mm_x_body, grid=(GRID,),
    in_specs=[_row_spec, _w_spec, _b_spec],
    out_specs=_row_spec, out_shape=_out1)

_mm_h = pl.pallas_call(
    _mm_h_body, grid=(GRID,),
    in_specs=[_row_spec, _pair_spec, _w_spec, _b_spec],
    out_specs=_row_spec, out_shape=_out1)

_fin = pl.pallas_call(
    _fin_body, grid=(GRID,),
    in_specs=[_row_spec, _pair_spec],
    out_specs=_row_spec, out_shape=_out1)


_mesh = plsc.VectorSubcoreMesh(core_axis_name="c", subcore_axis_name="s")


@functools.partial(
    pl.kernel, mesh=_mesh,
    out_type=jax.ShapeDtypeStruct((NC, NPAD, D), jnp.float32),
    scratch_types=[
        pltpu.VMEM((3, 3, 2, K), jnp.int32),
        pltpu.VMEM((K, D), jnp.float32),
        pltpu.VMEM((K, D), jnp.float32),
        pltpu.VMEM((K, D), jnp.float32),
        pltpu.VMEM_SHARED((NPAD, D), jnp.float32),
        pltpu.SemaphoreType.DMA,
        pltpu.SemaphoreType.DMA,
        pltpu.SemaphoreType.DMA,
        pltpu.SemaphoreType.DMA,
        pltpu.SemaphoreType.DMA,
    ])
def _sc_scatter(nbr_hbm, idx_hbm, zeros_hbm, out_hbm,
                bulk, rows0, rows1, rows2, acc, gsem0, gsem1, gsem2, ssem,
                isem):
    c = lax.axis_index("c")
    s = lax.axis_index("s")
    wid = s * NC + c
    r0 = s * RPT
    my_idx = idx_hbm.at[wid]
    rows = (rows0, rows1, rows2)
    gsems = (gsem0, gsem1, gsem2)
    pltpu.sync_copy(zeros_hbm.at[pl.ds(r0, RPT)], acc.at[pl.ds(r0, RPT)])
    plsc.subcore_barrier()

    pltpu.sync_copy(my_idx.at[0], bulk.at[0])
    pltpu.async_copy(nbr_hbm.at[bulk.at[0].at[0].at[0]], rows0, gsem0)
    pltpu.async_copy(nbr_hbm.at[bulk.at[0].at[1].at[0]], rows1, gsem1)
    pltpu.async_copy(my_idx.at[1], bulk.at[1], isem)

    def body(t, carry):
        cur = bulk.at[t % 3]
        nx1 = bulk.at[(t + 1) % 3]
        more = t + 1 < T

        pltpu.make_async_copy(nbr_hbm.at[cur.at[0].at[0]], rows0, gsem0).wait()

        @pl.when(t > 0)
        def _():
            pltpu.make_async_copy(rows2, acc.at[cur.at[0].at[1]], ssem).wait()

        @pl.when(more)
        def _():
            pltpu.make_async_copy(my_idx.at[t + 1], nx1, isem).wait()

        @pl.when(t + 2 < T)
        def _():
            pltpu.async_copy(my_idx.at[t + 2], bulk.at[(t + 2) % 3], isem)

        pltpu.async_copy(rows0, acc.at[cur.at[0].at[1]], ssem, add=True)
        pltpu.async_copy(nbr_hbm.at[cur.at[2].at[0]], rows2, gsem2)

        pltpu.make_async_copy(nbr_hbm.at[cur.at[1].at[0]], rows1, gsem1).wait()
        pltpu.make_async_copy(rows0, acc.at[cur.at[1].at[1]], ssem).wait()
        pltpu.async_copy(rows1, acc.at[cur.at[1].at[1]], ssem, add=True)

        @pl.when(more)
        def _():
            pltpu.async_copy(nbr_hbm.at[nx1.at[0].at[0]], rows0, gsem0)

        pltpu.make_async_copy(nbr_hbm.at[cur.at[2].at[0]], rows2, gsem2).wait()
        pltpu.make_async_copy(rows1, acc.at[cur.at[2].at[1]], ssem).wait()
        pltpu.async_copy(rows2, acc.at[cur.at[2].at[1]], ssem, add=True)

        @pl.when(more)
        def _():
            pltpu.async_copy(nbr_hbm.at[nx1.at[1].at[0]], rows1, gsem1)

        return carry

    lax.fori_loop(0, T, body, 0)
    pltpu.make_async_copy(rows2, acc.at[bulk.at[(T - 1) % 3].at[2].at[1]],
                          ssem).wait()
    plsc.subcore_barrier()
    pltpu.sync_copy(acc.at[pl.ds(r0, RPT)], out_hbm.at[c].at[pl.ds(r0, RPT)])


def kernel(features, edges, W0s, b0s, W1s, b1s):
    x = jnp.zeros((NPAD, D), jnp.float32).at[:N].set(features)
    src = edges[:, 0].astype(jnp.int32)
    dst = edges[:, 1].astype(jnp.int32)
    npad_msg = M_PAD - M
    pad_g = jnp.arange(npad_msg, dtype=jnp.int32) % N
    pad_s = N + jnp.arange(npad_msg, dtype=jnp.int32) % (NPAD - N)
    gidx = jnp.concatenate([dst, src, pad_g]).reshape(NW, T, 3, 1, K)
    sidx = jnp.concatenate([src, dst, pad_s]).reshape(NW, T, 3, 1, K)
    idx = jnp.concatenate([gidx, sidx], axis=3)
    zeros = jnp.zeros((NPAD, D), jnp.float32)
    b0r = b0s.reshape(4, 1, D)
    b1r = b1s.reshape(4, 1, D)

    nbr = _mm_x(x, W1s[0], b1r[0])
    p = _sc_scatter(nbr, idx, zeros)
    out = _mm_x(x, W0s[0], b0r[0])
    for k in (1, 2, 3):
        nbr = _mm_h(out, p, W1s[k], b1r[k])
        p2 = _sc_scatter(nbr, idx, zeros)
        out = _mm_h(out, p, W0s[k], b0r[k])
        p = p2
    y = _fin(out, p)
    return y[:N]


# speedup vs baseline: 1.0135x; 1.0135x over previous
import functools

import jax
import jax.numpy as jnp
from jax import lax
from jax.experimental import pallas as pl
from jax.experimental.pallas import tpu as pltpu
from jax.experimental.pallas import tpu_sc as plsc

N = 10000
D = 128
NPAD = 10240
E = 320000
M = 2 * E
NC = 2
NS = 16
NW = NC * NS
K = 120
T = 56
CHUNKS = 3 * T
MSG_PER_W = K * CHUNKS
M_PAD = MSG_PER_W * NW
RPT = NPAD // NS

BR = 2000
GRID = N // BR

_P = jax.lax.Precision.HIGHEST


def _mm_x_body(x_ref, w_ref, b_ref, out_ref):
    out_ref[...] = lax.dot_general(x_ref[...], w_ref[...],
                                   (((1,), (1,)), ((), ())),
                                   precision=_P) + b_ref[...]


def _mm_h_body(o_ref, p_ref, w_ref, b_ref, out_ref):
    h = jnp.maximum(o_ref[...] + p_ref[0] + p_ref[1], 0.0)
    out_ref[...] = lax.dot_general(h, w_ref[...], (((1,), (1,)), ((), ())),
                                   precision=_P) + b_ref[...]


def _fin_body(o_ref, p_ref, out_ref):
    out_ref[...] = o_ref[...] + p_ref[0] + p_ref[1]


_row_spec = pl.BlockSpec((BR, D), lambda i: (i, 0))
_pair_spec = pl.BlockSpec((2, BR, D), lambda i: (0, i, 0))
_w_spec = pl.BlockSpec((D, D), lambda i: (0, 0))
_b_spec = pl.BlockSpec((1, D), lambda i: (0, 0))
_out1 = jax.ShapeDtypeStruct((N, D), jnp.float32)

_mm_x = pl.pallas_call(
    _mm_x_body, grid=(GRID,),
    in_specs=[_row_spec, _w_spec, _b_spec],
    out_specs=_row_spec, out_shape=_out1)

_mm_h = pl.pallas_call(
    _mm_h_body, grid=(GRID,),
    in_specs=[_row_spec, _pair_spec, _w_spec, _b_spec],
    out_specs=_row_spec, out_shape=_out1)

_fin = pl.pallas_call(
    _fin_body, grid=(GRID,),
    in_specs=[_row_spec, _pair_spec],
    out_specs=_row_spec, out_shape=_out1)


_mesh = plsc.VectorSubcoreMesh(core_axis_name="c", subcore_axis_name="s")


@functools.partial(
    pl.kernel, mesh=_mesh,
    out_type=jax.ShapeDtypeStruct((NC, NPAD, D), jnp.float32),
    scratch_types=[
        pltpu.VMEM((3, 3, 2, K), jnp.int32),
        pltpu.VMEM((K, D), jnp.float32),
        pltpu.VMEM((K, D), jnp.float32),
        pltpu.VMEM((K, D), jnp.float32),
        pltpu.VMEM_SHARED((NPAD, D), jnp.float32),
        pltpu.SemaphoreType.DMA,
        pltpu.SemaphoreType.DMA,
        pltpu.SemaphoreType.DMA,
        pltpu.SemaphoreType.DMA,
        pltpu.SemaphoreType.DMA,
    ])
def _sc_scatter(nbr_hbm, idx_hbm, zeros_hbm, out_hbm,
                bulk, rows0, rows1, rows2, acc, gsem0, gsem1, gsem2, ssem,
                isem):
    c = lax.axis_index("c")
    s = lax.axis_index("s")
    wid = s * NC + c
    r0 = s * RPT
    my_idx = idx_hbm.at[wid]
    rows = (rows0, rows1, rows2)
    gsems = (gsem0, gsem1, gsem2)
    pltpu.sync_copy(zeros_hbm.at[pl.ds(r0, RPT)], acc.at[pl.ds(r0, RPT)])
    plsc.subcore_barrier()

    pltpu.sync_copy(my_idx.at[0], bulk.at[0])
    pltpu.async_copy(nbr_hbm.at[bulk.at[0].at[0].at[0]], rows0, gsem0)
    pltpu.async_copy(nbr_hbm.at[bulk.at[0].at[1].at[0]], rows1, gsem1)
    pltpu.async_copy(my_idx.at[1], bulk.at[1], isem)

    def body(t, carry):
        cur = bulk.at[t % 3]
        nx1 = bulk.at[(t + 1) % 3]
        more = t + 1 < T

        pltpu.make_async_copy(nbr_hbm.at[cur.at[0].at[0]], rows0, gsem0).wait()

        @pl.when(t > 0)
        def _():
            pltpu.make_async_copy(rows2, acc.at[cur.at[0].at[1]], ssem).wait()

        @pl.when(more)
        def _():
            pltpu.make_async_copy(my_idx.at[t + 1], nx1, isem).wait()

        @pl.when(t + 2 < T)
        def _():
            pltpu.async_copy(my_idx.at[t + 2], bulk.at[(t + 2) % 3], isem)

        pltpu.async_copy(rows0, acc.at[cur.at[0].at[1]], ssem, add=True)
        pltpu.async_copy(nbr_hbm.at[cur.at[2].at[0]], rows2, gsem2)

        pltpu.make_async_copy(nbr_hbm.at[cur.at[1].at[0]], rows1, gsem1).wait()
        pltpu.make_async_copy(rows0, acc.at[cur.at[1].at[1]], ssem).wait()
        pltpu.async_copy(rows1, acc.at[cur.at[1].at[1]], ssem, add=True)

        @pl.when(more)
        def _():
            pltpu.async_copy(nbr_hbm.at[nx1.at[0].at[0]], rows0, gsem0)

        pltpu.make_async_copy(nbr_hbm.at[cur.at[2].at[0]], rows2, gsem2).wait()
        pltpu.make_async_copy(rows1, acc.at[cur.at[2].at[1]], ssem).wait()
        pltpu.async_copy(rows2, acc.at[cur.at[2].at[1]], ssem, add=True)

        @pl.when(more)
        def _():
            pltpu.async_copy(nbr_hbm.at[nx1.at[1].at[0]], rows1, gsem1)

        return carry

    lax.fori_loop(0, T, body, 0)
    pltpu.make_async_copy(rows2, acc.at[bulk.at[(T - 1) % 3].at[2].at[1]],
                          ssem).wait()
    plsc.subcore_barrier()
    pltpu.sync_copy(acc.at[pl.ds(r0, RPT)], out_hbm.at[c].at[pl.ds(r0, RPT)])


def kernel(features, edges, W0s, b0s, W1s, b1s):
    src = edges[:, 0].astype(jnp.int32)
    dst = edges[:, 1].astype(jnp.int32)
    npad_msg = M_PAD - M
    pad_g = jnp.arange(npad_msg, dtype=jnp.int32) % N
    pad_s = N + jnp.arange(npad_msg, dtype=jnp.int32) % (NPAD - N)
    gidx = jnp.concatenate([dst, src, pad_g]).reshape(NW, T, 3, 1, K)
    sidx = jnp.concatenate([src, dst, pad_s]).reshape(NW, T, 3, 1, K)
    idx = jnp.concatenate([gidx, sidx], axis=3)
    zeros = jnp.zeros((NPAD, D), jnp.float32)
    b0r = b0s.reshape(4, 1, D)
    b1r = b1s.reshape(4, 1, D)

    nbr = _mm_x(features, W1s[0], b1r[0])
    p = _sc_scatter(nbr, idx, zeros)
    out = _mm_x(features, W0s[0], b0r[0])
    for k in (1, 2, 3):
        nbr = _mm_h(out, p, W1s[k], b1r[k])
        p2 = _sc_scatter(nbr, idx, zeros)
        out = _mm_h(out, p, W0s[k], b0r[k])
        p = p2
    return _fin(out, p)
